# baseline (device time: 173871 ns/iter reference)
import jax
import jax.numpy as jnp
from jax import lax
from jax.experimental import pallas as pl
from jax.experimental.pallas import tpu as pltpu

N_DEV = 4


def kernel(A, B):
    m, k = A.shape
    _, n = B.shape

    def body(a_ref, b_ref, out_ref, comm_ref, send_sems, recv_sems):
        my = lax.axis_index("i")
        left = (my - 1) % N_DEV
        right = (my + 1) % N_DEV

        barrier = pltpu.get_barrier_semaphore()
        for nbr in (left, right):
            pl.semaphore_signal(
                barrier, inc=1,
                device_id=(nbr,), device_id_type=pl.DeviceIdType.MESH,
            )
        pl.semaphore_wait(barrier, 2)

        a = a_ref[:, :].astype(jnp.bfloat16)
        b = b_ref[:, :].astype(jnp.bfloat16)
        partial = jnp.dot(a, b, preferred_element_type=jnp.float32)
        comm_ref[0, :, :] = partial.astype(jnp.bfloat16)
        acc = partial

        for h in range(N_DEV - 1):
            rdma = pltpu.make_async_remote_copy(
                src_ref=comm_ref.at[h],
                dst_ref=comm_ref.at[h + 1],
                send_sem=send_sems.at[h],
                recv_sem=recv_sems.at[h],
                device_id=(right,),
                device_id_type=pl.DeviceIdType.MESH,
            )
            rdma.start()
            rdma.wait()
            acc = acc + comm_ref[h + 1, :, :].astype(jnp.float32)

        out_ref[:, :] = acc.astype(jnp.bfloat16)

    return pl.pallas_call(
        body,
        out_shape=jax.ShapeDtypeStruct((m, n), jnp.bfloat16),
        in_specs=[
            pl.BlockSpec(memory_space=pltpu.VMEM),
            pl.BlockSpec(memory_space=pltpu.VMEM),
        ],
        out_specs=pl.BlockSpec(memory_space=pltpu.VMEM),
        scratch_shapes=[
            pltpu.VMEM((N_DEV, m, n), jnp.bfloat16),
            pltpu.SemaphoreType.DMA((N_DEV - 1,)),
            pltpu.SemaphoreType.DMA((N_DEV - 1,)),
        ],
        compiler_params=pltpu.CompilerParams(collective_id=0),
    )(A, B)


# device time: 59573 ns/iter; 2.9186x vs baseline; 2.9186x over previous
import jax
import jax.numpy as jnp
from jax import lax
from jax.experimental import pallas as pl
from jax.experimental.pallas import tpu as pltpu

N_DEV = 4


def kernel(A, B):
    m, k = A.shape
    _, n = B.shape
    half = m // 2
    sh = m // 4
    sq = m // 8

    def body(a_ref, b_ref, out_ref, p_ref, acc1_ref,
             rs1_send, rs1_recv, rs2_send, rs2_recv,
             send_sems, recv_sems):
        my = lax.axis_index("i")
        pa = my ^ 1
        pb = 3 - my

        barrier = pltpu.get_barrier_semaphore()
        for nbr in (pa, pb):
            pl.semaphore_signal(
                barrier, inc=1,
                device_id=(nbr,), device_id_type=pl.DeviceIdType.MESH,
            )
        pl.semaphore_wait(barrier, 2)

        a = a_ref[:, :].astype(jnp.bfloat16)
        b = b_ref[:, :].astype(jnp.bfloat16)
        p_ref[:, :] = jnp.dot(a, b, preferred_element_type=jnp.float32)

        bases = (0, half)
        cs = ((my & 1) ^ (my >> 1), my >> 1)
        ds = (my >> 1, my & 1)
        p1 = (pa, pb)
        p2 = (pb, pa)

        def xchg(src, dst, rnd, s, partner):
            return pltpu.make_async_remote_copy(
                src_ref=src, dst_ref=dst,
                send_sem=send_sems.at[rnd, s],
                recv_sem=recv_sems.at[rnd, s],
                device_id=(partner,),
                device_id_type=pl.DeviceIdType.MESH,
            )

        rdmas = []
        for s in range(2):
            send_rows = bases[s] + (1 - cs[s]) * sh
            rs1_send[s, :, :] = p_ref[pl.ds(send_rows, sh), :].astype(jnp.bfloat16)
            r = xchg(rs1_send.at[s], rs1_recv.at[s], 0, s, p1[s])
            r.start()
            rdmas.append(r)
        for r in rdmas:
            r.wait()
        for s in range(2):
            keep_rows = bases[s] + cs[s] * sh
            acc1_ref[s, :, :] = (
                p_ref[pl.ds(keep_rows, sh), :] + rs1_recv[s, :, :].astype(jnp.float32)
            )

        rdmas = []
        for s in range(2):
            rs2_send[s, :, :] = acc1_ref[
                s, pl.ds((1 - ds[s]) * sq, sq), :
            ].astype(jnp.bfloat16)
            r = xchg(rs2_send.at[s], rs2_recv.at[s], 1, s, p2[s])
            r.start()
            rdmas.append(r)
        for r in rdmas:
            r.wait()
        qrows = []
        for s in range(2):
            acc2 = (
                acc1_ref[s, pl.ds(ds[s] * sq, sq), :]
                + rs2_recv[s, :, :].astype(jnp.float32)
            )
            qr = bases[s] + cs[s] * sh + ds[s] * sq
            out_ref[pl.ds(qr, sq), :] = acc2.astype(jnp.bfloat16)
            qrows.append(qr)

        rdmas = []
        for s in range(2):
            sl = out_ref.at[pl.ds(qrows[s], sq), :]
            r = xchg(sl, sl, 2, s, p2[s])
            r.start()
            rdmas.append(r)
        for r in rdmas:
            r.wait()

        rdmas = []
        for s in range(2):
            hr = bases[s] + cs[s] * sh
            sl = out_ref.at[pl.ds(hr, sh), :]
            r = xchg(sl, sl, 3, s, p1[s])
            r.start()
            rdmas.append(r)
        for r in rdmas:
            r.wait()

    return pl.pallas_call(
        body,
        out_shape=jax.ShapeDtypeStruct((m, n), jnp.bfloat16),
        in_specs=[
            pl.BlockSpec(memory_space=pltpu.VMEM),
            pl.BlockSpec(memory_space=pltpu.VMEM),
        ],
        out_specs=pl.BlockSpec(memory_space=pltpu.VMEM),
        scratch_shapes=[
            pltpu.VMEM((m, n), jnp.float32),
            pltpu.VMEM((2, sh, n), jnp.float32),
            pltpu.VMEM((2, sh, n), jnp.bfloat16),
            pltpu.VMEM((2, sh, n), jnp.bfloat16),
            pltpu.VMEM((2, sq, n), jnp.bfloat16),
            pltpu.VMEM((2, sq, n), jnp.bfloat16),
            pltpu.SemaphoreType.DMA((4, 2)),
            pltpu.SemaphoreType.DMA((4, 2)),
        ],
        compiler_params=pltpu.CompilerParams(collective_id=0),
    )(A, B)


# device time: 57208 ns/iter; 3.0393x vs baseline; 1.0413x over previous
import jax
import jax.numpy as jnp
from jax import lax
from jax.experimental import pallas as pl
from jax.experimental.pallas import tpu as pltpu

N_DEV = 4


def kernel(A, B):
    m, k = A.shape
    _, n = B.shape
    half = m // 2
    sh = m // 4
    sq = m // 8

    def body(a_ref, b_ref, out_ref, acc1_ref,
             rs1_send, rs1_recv, rs2_send, rs2_recv,
             send_sems, recv_sems):
        my = lax.axis_index("i")
        pa = my ^ 1
        pb = 3 - my

        barrier = pltpu.get_barrier_semaphore()
        for nbr in (pa, pb):
            pl.semaphore_signal(
                barrier, inc=1,
                device_id=(nbr,), device_id_type=pl.DeviceIdType.MESH,
            )
        pl.semaphore_wait(barrier, 2)

        bases = (0, half)
        cs = ((my & 1) ^ (my >> 1), my >> 1)
        ds = (my >> 1, my & 1)
        p1 = (pa, pb)
        p2 = (pb, pa)

        def xchg(src, dst, rnd, s, partner):
            return pltpu.make_async_remote_copy(
                src_ref=src, dst_ref=dst,
                send_sem=send_sems.at[rnd, s],
                recv_sem=recv_sems.at[rnd, s],
                device_id=(partner,),
                device_id_type=pl.DeviceIdType.MESH,
            )

        b = b_ref[:, :].astype(jnp.bfloat16)
        inflight = []

        rs1 = []
        for s in range(2):
            send_rows = bases[s] + (1 - cs[s]) * sh
            a_s = a_ref[pl.ds(send_rows, sh), :].astype(jnp.bfloat16)
            rs1_send[s, :, :] = jnp.dot(
                a_s, b, preferred_element_type=jnp.float32
            ).astype(jnp.bfloat16)
            r = xchg(rs1_send.at[s], rs1_recv.at[s], 0, s, p1[s])
            r.start()
            rs1.append(r)
        inflight += rs1

        for s in range(2):
            keep_rows = bases[s] + cs[s] * sh
            a_s = a_ref[pl.ds(keep_rows, sh), :].astype(jnp.bfloat16)
            acc1_ref[s, :, :] = jnp.dot(a_s, b, preferred_element_type=jnp.float32)

        for s in range(2):
            rs1[s].wait_recv()
            acc1_ref[s, :, :] = acc1_ref[s, :, :] + rs1_recv[s, :, :].astype(
                jnp.float32
            )

        rs2 = []
        for s in range(2):
            rs2_send[s, :, :] = acc1_ref[
                s, pl.ds((1 - ds[s]) * sq, sq), :
            ].astype(jnp.bfloat16)
            r = xchg(rs2_send.at[s], rs2_recv.at[s], 1, s, p2[s])
            r.start()
            rs2.append(r)
        inflight += rs2
        qrows = []
        for s in range(2):
            rs2[s].wait_recv()
            acc2 = (
                acc1_ref[s, pl.ds(ds[s] * sq, sq), :]
                + rs2_recv[s, :, :].astype(jnp.float32)
            )
            qr = bases[s] + cs[s] * sh + ds[s] * sq
            out_ref[pl.ds(qr, sq), :] = acc2.astype(jnp.bfloat16)
            qrows.append(qr)

        ag1 = []
        for s in range(2):
            sl = out_ref.at[pl.ds(qrows[s], sq), :]
            r = xchg(sl, sl, 2, s, p2[s])
            r.start()
            ag1.append(r)
        inflight += ag1
        for r in ag1:
            r.wait_recv()

        ag2 = []
        for s in range(2):
            hr = bases[s] + cs[s] * sh
            sl = out_ref.at[pl.ds(hr, sh), :]
            r = xchg(sl, sl, 3, s, p1[s])
            r.start()
            ag2.append(r)
        inflight += ag2
        for r in ag2:
            r.wait_recv()

        for r in inflight:
            r.wait_send()

    return pl.pallas_call(
        body,
        out_shape=jax.ShapeDtypeStruct((m, n), jnp.bfloat16),
        in_specs=[
            pl.BlockSpec(memory_space=pltpu.VMEM),
            pl.BlockSpec(memory_space=pltpu.VMEM),
        ],
        out_specs=pl.BlockSpec(memory_space=pltpu.VMEM),
        scratch_shapes=[
            pltpu.VMEM((2, sh, n), jnp.float32),
            pltpu.VMEM((2, sh, n), jnp.bfloat16),
            pltpu.VMEM((2, sh, n), jnp.bfloat16),
            pltpu.VMEM((2, sq, n), jnp.bfloat16),
            pltpu.VMEM((2, sq, n), jnp.bfloat16),
            pltpu.SemaphoreType.DMA((4, 2)),
            pltpu.SemaphoreType.DMA((4, 2)),
        ],
        compiler_params=pltpu.CompilerParams(collective_id=0),
    )(A, B)


# device time: 55738 ns/iter; 3.1194x vs baseline; 1.0264x over previous
import jax
import jax.numpy as jnp
from jax import lax
from jax.experimental import pallas as pl
from jax.experimental.pallas import tpu as pltpu

N_DEV = 4

RS1A, RS1B, RS2, AG1, AG2A, AG2B = range(6)


def kernel(A, B):
    m, k = A.shape
    _, n = B.shape
    half = m // 2
    sh = m // 4
    sq = m // 8

    def body(a_ref, b_ref, out_ref, acc1_ref,
             rs1_send, rs1a_recv, rs1b_recv, rs2_send, rs2_recv,
             send_sems, recv_sems):
        my = lax.axis_index("i")
        pa = my ^ 1
        pb = 3 - my

        barrier = pltpu.get_barrier_semaphore()
        for nbr in (pa, pb):
            pl.semaphore_signal(
                barrier, inc=1,
                device_id=(nbr,), device_id_type=pl.DeviceIdType.MESH,
            )
        pl.semaphore_wait(barrier, 2)

        bases = (0, half)
        cs = ((my & 1) ^ (my >> 1), my >> 1)
        ds = (my >> 1, my & 1)
        p1 = (pa, pb)
        p2 = (pb, pa)
        first_q = (1 - ds[0], ds[1])

        def xchg(src, dst, ph, s, partner):
            return pltpu.make_async_remote_copy(
                src_ref=src, dst_ref=dst,
                send_sem=send_sems.at[ph, s],
                recv_sem=recv_sems.at[ph, s],
                device_id=(partner,),
                device_id_type=pl.DeviceIdType.MESH,
            )

        b = b_ref[:, :].astype(jnp.bfloat16)
        inflight = []

        rs1a, rs1b = [], []
        for s in range(2):
            send_rows = bases[s] + (1 - cs[s]) * sh
            a_s = a_ref[pl.ds(send_rows, sh), :].astype(jnp.bfloat16)
            rs1_send[s, :, :] = jnp.dot(
                a_s, b, preferred_element_type=jnp.float32
            ).astype(jnp.bfloat16)
            fq = first_q[s]
            ra = xchg(rs1_send.at[s, pl.ds(fq * sq, sq), :],
                      rs1a_recv.at[s], RS1A, s, p1[s])
            ra.start()
            rb = xchg(rs1_send.at[s, pl.ds((1 - fq) * sq, sq), :],
                      rs1b_recv.at[s], RS1B, s, p1[s])
            rb.start()
            rs1a.append(ra)
            rs1b.append(rb)
        inflight += rs1a + rs1b

        for s in range(2):
            keep_rows = bases[s] + cs[s] * sh
            a_s = a_ref[pl.ds(keep_rows, sh), :].astype(jnp.bfloat16)
            acc1_ref[s, :, :] = jnp.dot(a_s, b, preferred_element_type=jnp.float32)

        rs2 = []
        for s in range(2):
            rs1a[s].wait_recv()
            rs2_send[s, :, :] = (
                acc1_ref[s, pl.ds((1 - ds[s]) * sq, sq), :]
                + rs1a_recv[s, :, :].astype(jnp.float32)
            ).astype(jnp.bfloat16)
            r = xchg(rs2_send.at[s], rs2_recv.at[s], RS2, s, p2[s])
            r.start()
            rs2.append(r)
        inflight += rs2

        qrows = []
        ag1, ag2a = [], []
        for s in range(2):
            rs1b[s].wait_recv()
            rs2[s].wait_recv()
            acc2 = (
                acc1_ref[s, pl.ds(ds[s] * sq, sq), :]
                + rs1b_recv[s, :, :].astype(jnp.float32)
                + rs2_recv[s, :, :].astype(jnp.float32)
            )
            qr = bases[s] + cs[s] * sh + ds[s] * sq
            out_ref[pl.ds(qr, sq), :] = acc2.astype(jnp.bfloat16)
            qrows.append(qr)
            sl = out_ref.at[pl.ds(qr, sq), :]
            r1 = xchg(sl, sl, AG1, s, p2[s])
            r1.start()
            r2 = xchg(sl, sl, AG2A, s, p1[s])
            r2.start()
            ag1.append(r1)
            ag2a.append(r2)
        inflight += ag1 + ag2a

        ag2b = []
        for s in range(2):
            ag1[s].wait_recv()
            fr = bases[s] + cs[s] * sh + (1 - ds[s]) * sq
            sl = out_ref.at[pl.ds(fr, sq), :]
            r = xchg(sl, sl, AG2B, s, p1[s])
            r.start()
            ag2b.append(r)
        inflight += ag2b

        for s in range(2):
            ag2a[s].wait_recv()
            ag2b[s].wait_recv()

        for r in inflight:
            r.wait_send()

    return pl.pallas_call(
        body,
        out_shape=jax.ShapeDtypeStruct((m, n), jnp.bfloat16),
        in_specs=[
            pl.BlockSpec(memory_space=pltpu.VMEM),
            pl.BlockSpec(memory_space=pltpu.VMEM),
        ],
        out_specs=pl.BlockSpec(memory_space=pltpu.VMEM),
        scratch_shapes=[
            pltpu.VMEM((2, sh, n), jnp.float32),
            pltpu.VMEM((2, sh, n), jnp.bfloat16),
            pltpu.VMEM((2, sq, n), jnp.bfloat16),
            pltpu.VMEM((2, sq, n), jnp.bfloat16),
            pltpu.VMEM((2, sq, n), jnp.bfloat16),
            pltpu.VMEM((2, sq, n), jnp.bfloat16),
            pltpu.SemaphoreType.DMA((6, 2)),
            pltpu.SemaphoreType.DMA((6, 2)),
        ],
        compiler_params=pltpu.CompilerParams(collective_id=0),
    )(A, B)


# device time: 51528 ns/iter; 3.3743x vs baseline; 1.0817x over previous
import jax
import jax.numpy as jnp
from jax import lax
from jax.experimental import pallas as pl
from jax.experimental.pallas import tpu as pltpu

N_DEV = 4

RS1A0, RS1A1, RS1B, RS2_0, RS2_1, AG1_0, AG1_1, AG2A0, AG2A1, AG2B0, AG2B1 = (
    range(11)
)


def kernel(A, B):
    m, k = A.shape
    _, n = B.shape
    half = m // 2
    sh = m // 4
    sq = m // 8
    sqh = sq // 2

    def body(a_ref, b_ref, out_ref, acc1_ref,
             rs1_send, rs1a_recv, rs1b_recv, rs2_send, rs2_recv,
             send_sems, recv_sems):
        my = lax.axis_index("i")
        pa = my ^ 1
        pb = 3 - my

        barrier = pltpu.get_barrier_semaphore()
        for nbr in (pa, pb):
            pl.semaphore_signal(
                barrier, inc=1,
                device_id=(nbr,), device_id_type=pl.DeviceIdType.MESH,
            )
        pl.semaphore_wait(barrier, 2)

        bases = (0, half)
        cs = ((my & 1) ^ (my >> 1), my >> 1)
        ds = (my >> 1, my & 1)
        p1 = (pa, pb)
        p2 = (pb, pa)
        first_q = (1 - ds[0], ds[1])

        def xchg(src, dst, ph, s, partner):
            return pltpu.make_async_remote_copy(
                src_ref=src, dst_ref=dst,
                send_sem=send_sems.at[ph, s],
                recv_sem=recv_sems.at[ph, s],
                device_id=(partner,),
                device_id_type=pl.DeviceIdType.MESH,
            )

        b = b_ref[:, :].astype(jnp.bfloat16)
        inflight = []

        def qdot(rows):
            a_s = a_ref[pl.ds(rows, sq), :].astype(jnp.bfloat16)
            return jnp.dot(a_s, b, preferred_element_type=jnp.float32)

        rs1a = [[None, None], [None, None]]
        rs1b = []
        for s in range(2):
            fq = first_q[s]
            rows = bases[s] + (1 - cs[s]) * sh + fq * sq
            rs1_send[s, pl.ds(fq * sq, sq), :] = qdot(rows).astype(jnp.bfloat16)
            for j in range(2):
                r = xchg(rs1_send.at[s, pl.ds(fq * sq + j * sqh, sqh), :],
                         rs1a_recv.at[s, pl.ds(j * sqh, sqh), :],
                         RS1A0 + j, s, p1[s])
                r.start()
                rs1a[s][j] = r
                inflight.append(r)
        for s in range(2):
            fq = first_q[s]
            rows = bases[s] + (1 - cs[s]) * sh + (1 - fq) * sq
            rs1_send[s, pl.ds((1 - fq) * sq, sq), :] = qdot(rows).astype(
                jnp.bfloat16
            )
            r = xchg(rs1_send.at[s, pl.ds((1 - fq) * sq, sq), :],
                     rs1b_recv.at[s], RS1B, s, p1[s])
            r.start()
            rs1b.append(r)
            inflight.append(r)

        for s in range(2):
            rows = bases[s] + cs[s] * sh + (1 - ds[s]) * sq
            acc1_ref[s, pl.ds((1 - ds[s]) * sq, sq), :] = qdot(rows)

        rs2 = [[None, None], [None, None]]
        for j in range(2):
            for s in range(2):
                rs1a[s][j].wait_recv()
                rs2_send[s, pl.ds(j * sqh, sqh), :] = (
                    acc1_ref[s, pl.ds((1 - ds[s]) * sq + j * sqh, sqh), :]
                    + rs1a_recv[s, pl.ds(j * sqh, sqh), :].astype(jnp.float32)
                ).astype(jnp.bfloat16)
                r = xchg(rs2_send.at[s, pl.ds(j * sqh, sqh), :],
                         rs2_recv.at[s, pl.ds(j * sqh, sqh), :],
                         RS2_0 + j, s, p2[s])
                r.start()
                rs2[s][j] = r
                inflight.append(r)

        for s in range(2):
            rows = bases[s] + cs[s] * sh + ds[s] * sq
            acc1_ref[s, pl.ds(ds[s] * sq, sq), :] = qdot(rows)
        for s in range(2):
            rs1b[s].wait_recv()
            acc1_ref[s, pl.ds(ds[s] * sq, sq), :] = (
                acc1_ref[s, pl.ds(ds[s] * sq, sq), :]
                + rs1b_recv[s, :, :].astype(jnp.float32)
            )

        qrows = []
        for s in range(2):
            qrows.append(bases[s] + cs[s] * sh + ds[s] * sq)
        ag1 = [[None, None], [None, None]]
        ag2a = [[None, None], [None, None]]
        for j in range(2):
            for s in range(2):
                rs2[s][j].wait_recv()
                acc2 = (
                    acc1_ref[s, pl.ds(ds[s] * sq + j * sqh, sqh), :]
                    + rs2_recv[s, pl.ds(j * sqh, sqh), :].astype(jnp.float32)
                )
                out_ref[pl.ds(qrows[s] + j * sqh, sqh), :] = acc2.astype(
                    jnp.bfloat16
                )
                sl = out_ref.at[pl.ds(qrows[s] + j * sqh, sqh), :]
                r1 = xchg(sl, sl, AG1_0 + j, s, p2[s])
                r1.start()
                r2 = xchg(sl, sl, AG2A0 + j, s, p1[s])
                r2.start()
                ag1[s][j] = r1
                ag2a[s][j] = r2
                inflight += [r1, r2]

        ag2b = [[None, None], [None, None]]
        for j in range(2):
            for s in range(2):
                ag1[s][j].wait_recv()
                fr = bases[s] + cs[s] * sh + (1 - ds[s]) * sq + j * sqh
                sl = out_ref.at[pl.ds(fr, sqh), :]
                r = xchg(sl, sl, AG2B0 + j, s, p1[s])
                r.start()
                ag2b[s][j] = r
                inflight.append(r)

        for s in range(2):
            for j in range(2):
                ag2a[s][j].wait_recv()
                ag2b[s][j].wait_recv()

        for r in inflight:
            r.wait_send()

    return pl.pallas_call(
        body,
        out_shape=jax.ShapeDtypeStruct((m, n), jnp.bfloat16),
        in_specs=[
            pl.BlockSpec(memory_space=pltpu.VMEM),
            pl.BlockSpec(memory_space=pltpu.VMEM),
        ],
        out_specs=pl.BlockSpec(memory_space=pltpu.VMEM),
        scratch_shapes=[
            pltpu.VMEM((2, sh, n), jnp.float32),
            pltpu.VMEM((2, sh, n), jnp.bfloat16),
            pltpu.VMEM((2, sq, n), jnp.bfloat16),
            pltpu.VMEM((2, sq, n), jnp.bfloat16),
            pltpu.VMEM((2, sq, n), jnp.bfloat16),
            pltpu.VMEM((2, sq, n), jnp.bfloat16),
            pltpu.SemaphoreType.DMA((11, 2)),
            pltpu.SemaphoreType.DMA((11, 2)),
        ],
        compiler_params=pltpu.CompilerParams(collective_id=0),
    )(A, B)


# device time: 50593 ns/iter; 3.4367x vs baseline; 1.0185x over previous
import jax
import jax.numpy as jnp
from jax import lax
from jax.experimental import pallas as pl
from jax.experimental.pallas import tpu as pltpu

N_DEV = 4

RS1A0, RS1A1, RS1B, RS2_0, RS2_1, AG1_0, AG1_1, AG2A0, AG2A1, AG2B0, AG2B1 = (
    range(11)
)


def kernel(A, B):
    m, k = A.shape
    _, n = B.shape
    half = m // 2
    sh = m // 4
    sq = m // 8
    sqh = sq // 2

    def body(a_ref, b_ref, out_hbm, outv, acc1_ref,
             rs1_send, rs1a_recv, rs1b_recv, rs2_send, rs2_recv,
             send_sems, recv_sems, copy_sems):
        out_ref = outv
        my = lax.axis_index("i")
        pa = my ^ 1
        pb = 3 - my

        barrier = pltpu.get_barrier_semaphore()
        for nbr in (pa, pb):
            pl.semaphore_signal(
                barrier, inc=1,
                device_id=(nbr,), device_id_type=pl.DeviceIdType.MESH,
            )
        pl.semaphore_wait(barrier, 2)

        bases = (0, half)
        cs = ((my & 1) ^ (my >> 1), my >> 1)
        ds = (my >> 1, my & 1)
        p1 = (pa, pb)
        p2 = (pb, pa)
        first_q = (1 - ds[0], ds[1])

        def xchg(src, dst, ph, s, partner):
            return pltpu.make_async_remote_copy(
                src_ref=src, dst_ref=dst,
                send_sem=send_sems.at[ph, s],
                recv_sem=recv_sems.at[ph, s],
                device_id=(partner,),
                device_id_type=pl.DeviceIdType.MESH,
            )

        b = b_ref[:, :].astype(jnp.bfloat16)
        inflight = []

        def qdot(rows):
            a_s = a_ref[pl.ds(rows, sq), :].astype(jnp.bfloat16)
            return jnp.dot(a_s, b, preferred_element_type=jnp.float32)

        rs1a = [[None, None], [None, None]]
        rs1b = []
        for s in range(2):
            fq = first_q[s]
            rows = bases[s] + (1 - cs[s]) * sh + fq * sq
            rs1_send[s, pl.ds(fq * sq, sq), :] = qdot(rows).astype(jnp.bfloat16)
            for j in range(2):
                r = xchg(rs1_send.at[s, pl.ds(fq * sq + j * sqh, sqh), :],
                         rs1a_recv.at[s, pl.ds(j * sqh, sqh), :],
                         RS1A0 + j, s, p1[s])
                r.start()
                rs1a[s][j] = r
                inflight.append(r)
        for s in range(2):
            fq = first_q[s]
            rows = bases[s] + (1 - cs[s]) * sh + (1 - fq) * sq
            rs1_send[s, pl.ds((1 - fq) * sq, sq), :] = qdot(rows).astype(
                jnp.bfloat16
            )
            r = xchg(rs1_send.at[s, pl.ds((1 - fq) * sq, sq), :],
                     rs1b_recv.at[s], RS1B, s, p1[s])
            r.start()
            rs1b.append(r)
            inflight.append(r)

        for s in range(2):
            rows = bases[s] + cs[s] * sh + (1 - ds[s]) * sq
            acc1_ref[s, pl.ds((1 - ds[s]) * sq, sq), :] = qdot(rows)

        rs2 = [[None, None], [None, None]]
        for j in range(2):
            for s in range(2):
                rs1a[s][j].wait_recv()
                rs2_send[s, pl.ds(j * sqh, sqh), :] = (
                    acc1_ref[s, pl.ds((1 - ds[s]) * sq + j * sqh, sqh), :]
                    + rs1a_recv[s, pl.ds(j * sqh, sqh), :].astype(jnp.float32)
                ).astype(jnp.bfloat16)
                r = xchg(rs2_send.at[s, pl.ds(j * sqh, sqh), :],
                         rs2_recv.at[s, pl.ds(j * sqh, sqh), :],
                         RS2_0 + j, s, p2[s])
                r.start()
                rs2[s][j] = r
                inflight.append(r)

        for s in range(2):
            rows = bases[s] + cs[s] * sh + ds[s] * sq
            acc1_ref[s, pl.ds(ds[s] * sq, sq), :] = qdot(rows)
        for s in range(2):
            rs1b[s].wait_recv()
            acc1_ref[s, pl.ds(ds[s] * sq, sq), :] = (
                acc1_ref[s, pl.ds(ds[s] * sq, sq), :]
                + rs1b_recv[s, :, :].astype(jnp.float32)
            )

        qrows = []
        for s in range(2):
            qrows.append(bases[s] + cs[s] * sh + ds[s] * sq)
        ag1 = [[None, None], [None, None]]
        ag2a = [[None, None], [None, None]]
        for j in range(2):
            for s in range(2):
                rs2[s][j].wait_recv()
                acc2 = (
                    acc1_ref[s, pl.ds(ds[s] * sq + j * sqh, sqh), :]
                    + rs2_recv[s, pl.ds(j * sqh, sqh), :].astype(jnp.float32)
                )
                out_ref[pl.ds(qrows[s] + j * sqh, sqh), :] = acc2.astype(
                    jnp.bfloat16
                )
                sl = out_ref.at[pl.ds(qrows[s] + j * sqh, sqh), :]
                r1 = xchg(sl, sl, AG1_0 + j, s, p2[s])
                r1.start()
                r2 = xchg(sl, sl, AG2A0 + j, s, p1[s])
                r2.start()
                ag1[s][j] = r1
                ag2a[s][j] = r2
                inflight += [r1, r2]

        ag2b = [[None, None], [None, None]]
        for j in range(2):
            for s in range(2):
                ag1[s][j].wait_recv()
                fr = bases[s] + cs[s] * sh + (1 - ds[s]) * sq + j * sqh
                sl = out_ref.at[pl.ds(fr, sqh), :]
                r = xchg(sl, sl, AG2B0 + j, s, p1[s])
                r.start()
                ag2b[s][j] = r
                inflight.append(r)

        keep_copies = []
        for s in range(2):
            hr = bases[s] + cs[s] * sh
            cp = pltpu.make_async_copy(
                outv.at[pl.ds(hr, sh), :],
                out_hbm.at[pl.ds(hr, sh), :],
                copy_sems.at[0, s],
            )
            cp.start()
            keep_copies.append(cp)

        other_copies = []
        for s in range(2):
            for j in range(2):
                ag2a[s][j].wait_recv()
                ag2b[s][j].wait_recv()
            hr = bases[s] + (1 - cs[s]) * sh
            cp = pltpu.make_async_copy(
                outv.at[pl.ds(hr, sh), :],
                out_hbm.at[pl.ds(hr, sh), :],
                copy_sems.at[1, s],
            )
            cp.start()
            other_copies.append(cp)
        for cp in keep_copies + other_copies:
            cp.wait()

        for r in inflight:
            r.wait_send()

    return pl.pallas_call(
        body,
        out_shape=jax.ShapeDtypeStruct((m, n), jnp.bfloat16),
        in_specs=[
            pl.BlockSpec(memory_space=pltpu.VMEM),
            pl.BlockSpec(memory_space=pltpu.VMEM),
        ],
        out_specs=pl.BlockSpec(memory_space=pltpu.HBM),
        scratch_shapes=[
            pltpu.VMEM((m, n), jnp.bfloat16),
            pltpu.VMEM((2, sh, n), jnp.float32),
            pltpu.VMEM((2, sh, n), jnp.bfloat16),
            pltpu.VMEM((2, sq, n), jnp.bfloat16),
            pltpu.VMEM((2, sq, n), jnp.bfloat16),
            pltpu.VMEM((2, sq, n), jnp.bfloat16),
            pltpu.VMEM((2, sq, n), jnp.bfloat16),
            pltpu.SemaphoreType.DMA((11, 2)),
            pltpu.SemaphoreType.DMA((11, 2)),
            pltpu.SemaphoreType.DMA((2, 2)),
        ],
        compiler_params=pltpu.CompilerParams(collective_id=0),
    )(A, B)
